# baseline (device time: 31236 ns/iter reference)
import functools

import jax
import jax.numpy as jnp
from jax import lax
from jax.experimental import pallas as pl
from jax.experimental.pallas import tpu as pltpu

N_Z = 4
P = 2


def kernel(x):
    m_per, n = x.shape
    half = m_per // 2
    quart = m_per // 4
    piece = quart // P

    def body(x_ref, out_ref,
             send_up, recv_up, send_dn, recv_dn,
             sx, rx, sy, ry, sd, rd):
        my_x = lax.axis_index("x")
        my_y = lax.axis_index("y")
        my_z = lax.axis_index("z")
        has_up = my_z < N_Z - 1
        has_dn = my_z > 0

        def rows(o, xx, yy, p):
            return o * m_per + xx * half + yy * quart + p * piece

        def copy(row0, ssem, rsem, slot, dev):
            return pltpu.make_async_remote_copy(
                src_ref=out_ref.at[pl.ds(row0, piece), :],
                dst_ref=out_ref.at[pl.ds(row0, piece), :],
                send_sem=ssem.at[slot],
                recv_sem=rsem.at[slot],
                device_id=dev,
                device_id_type=pl.DeviceIdType.MESH,
            )

        def z_copy(o, p, dz, ssem, rsem):
            return copy(rows(o, my_x, my_y, p), ssem, rsem, o * P + p,
                        (my_x, my_y, my_z + dz))

        def x_own(o, p):
            return copy(rows(o, my_x, my_y, p), sx, rx, o * P + p,
                        (1 - my_x, my_y, my_z))

        def y_own(o, p):
            return copy(rows(o, my_x, my_y, p), sy, ry, o * P + p,
                        (my_x, 1 - my_y, my_z))

        def d_own(o, p):
            return copy(rows(o, my_x, my_y, p), sd, rd, o * P + p,
                        (1 - my_x, 1 - my_y, my_z))

        def xy_recv(o, p, xx, yy, ssem, rsem, dev):
            return copy(rows(o, xx, yy, p), ssem, rsem, o * P + p, dev)

        def pushes(o, p):
            x_own(o, p).start()
            y_own(o, p).start()
            d_own(o, p).start()

        def neighbor_signal(sem):
            for dev in ((1 - my_x, my_y, my_z), (my_x, 1 - my_y, my_z),
                        (1 - my_x, 1 - my_y, my_z)):
                pl.semaphore_signal(
                    sem, inc=1, device_id=dev,
                    device_id_type=pl.DeviceIdType.MESH,
                )
            @pl.when(has_up)
            def _():
                pl.semaphore_signal(
                    sem, inc=1, device_id=(my_x, my_y, my_z + 1),
                    device_id_type=pl.DeviceIdType.MESH,
                )
            @pl.when(has_dn)
            def _():
                pl.semaphore_signal(
                    sem, inc=1, device_id=(my_x, my_y, my_z - 1),
                    device_id_type=pl.DeviceIdType.MESH,
                )

        def neighbor_wait(sem):
            is_middle = jnp.logical_and(has_up, has_dn)
            @pl.when(is_middle)
            def _():
                pl.semaphore_wait(sem, 5)
            @pl.when(jnp.logical_not(is_middle))
            def _():
                pl.semaphore_wait(sem, 4)

        barrier_sem = pltpu.get_barrier_semaphore()
        neighbor_signal(barrier_sem)
        neighbor_wait(barrier_sem)

        out_ref[pl.ds(my_z * m_per, m_per), :] = x_ref[:, :]

        for p in range(P):
            @pl.when(has_up)
            def _():
                z_copy(my_z, p, 1, send_up, recv_up).start()
            @pl.when(has_dn)
            def _():
                z_copy(my_z, p, -1, send_dn, recv_dn).start()

        for s in range(1, N_Z):
            o_ur = my_z - s
            o_dr = my_z + s
            for p in range(P):
                @pl.when(o_ur >= 0)
                def _():
                    z_copy(o_ur, p, 1, send_up, recv_up).wait_recv()
                @pl.when(jnp.logical_and(o_ur >= 0, has_up))
                def _():
                    z_copy(o_ur, p, 1, send_up, recv_up).start()
                @pl.when(o_ur >= 0)
                def _():
                    pushes(o_ur, p)

                @pl.when(o_dr <= N_Z - 1)
                def _():
                    z_copy(o_dr, p, -1, send_dn, recv_dn).wait_recv()
                @pl.when(jnp.logical_and(o_dr <= N_Z - 1, has_dn))
                def _():
                    z_copy(o_dr, p, -1, send_dn, recv_dn).start()
                @pl.when(o_dr <= N_Z - 1)
                def _():
                    pushes(o_dr, p)

        for s in range(1, N_Z):
            for o in (my_z - s, my_z + s):
                for p in range(P):
                    @pl.when(jnp.logical_and(o >= 0, o <= N_Z - 1))
                    def _():
                        xy_recv(o, p, 1 - my_x, my_y, sx, rx,
                                (1 - my_x, my_y, my_z)).wait_recv()
                        xy_recv(o, p, my_x, 1 - my_y, sy, ry,
                                (my_x, 1 - my_y, my_z)).wait_recv()
                        xy_recv(o, p, 1 - my_x, 1 - my_y, sd, rd,
                                (1 - my_x, 1 - my_y, my_z)).wait_recv()

        for p in range(P):
            for o in range(N_Z):
                up_used = jnp.logical_and(has_up, o <= my_z)
                dn_used = jnp.logical_and(has_dn, o >= my_z)

                @pl.when(up_used)
                def _():
                    z_copy(o, p, 1, send_up, recv_up).wait_send()

                @pl.when(dn_used)
                def _():
                    z_copy(o, p, -1, send_dn, recv_dn).wait_send()

                @pl.when(o != my_z)
                def _():
                    x_own(o, p).wait_send()
                    y_own(o, p).wait_send()
                    d_own(o, p).wait_send()

        @functools.partial(
            pl.run_scoped, second_barrier=pltpu.SemaphoreType.REGULAR
        )
        def _(second_barrier):
            neighbor_signal(second_barrier)
            neighbor_wait(second_barrier)

    dma = pltpu.SemaphoreType.DMA((N_Z * P,))
    return pl.pallas_call(
        body,
        out_shape=jax.ShapeDtypeStruct((N_Z * m_per, n), x.dtype),
        in_specs=[pl.BlockSpec(memory_space=pltpu.VMEM)],
        out_specs=pl.BlockSpec(memory_space=pltpu.VMEM),
        scratch_shapes=[dma] * 10,
        compiler_params=pltpu.CompilerParams(collective_id=0),
    )(x)
